# baseline (speedup 1.0000x reference)
import functools

import jax
import jax.numpy as jnp
from jax import lax
from jax.experimental import pallas as pl
from jax.experimental.pallas import tpu as pltpu
from jax.experimental.pallas import tpu_sc as plsc

N = 50000
C = 5
G = 2
L = 4
M = 32
CG = C * G
LANES = 16
NC, NS = 2, 16
NW = NC * NS
EDGE_BLK = 1280
RPB = EDGE_BLK // 128
ACC_ROWS = 51200
STRIPE = ACC_ROWS // NS
ZROWS = 640
NB_TC = 10
NB_ROWS = N // NB_TC


def _pre_body(e, erows_blk, ph_ref, a_ref, bias_ref, x_ref, bst_ref,
              ei_ref, pos_ref, wtab_ref, bx_ref,
              srcp_ref, cidx_ref):
    w = jnp.dot(ph_ref[...], a_ref[...], preferred_element_type=jnp.float32)
    w = w + bias_ref[...]
    for p in range(L):
        wtab_ref[p::L, :] = w[:, p * LANES:(p + 1) * LANES]
    onehot = (x_ref[...] == lax.broadcasted_iota(jnp.int32, (1, M), 1)
              ).astype(jnp.float32)
    bx_ref[...] = jnp.dot(onehot, bst_ref[...],
                          preferred_element_type=jnp.float32)
    i = pl.program_id(0)
    shape = (erows_blk, 128)
    src_blk = ei_ref[0].reshape(shape)
    dst_blk = ei_ref[1].reshape(shape)
    pos_blk = pos_ref[...].reshape(shape)
    ge = ((i * erows_blk + lax.broadcasted_iota(jnp.int32, shape, 0)) * 128
          + lax.broadcasted_iota(jnp.int32, shape, 1))
    real = ge < e
    srcp_ref[...] = jnp.where(real, src_blk, N + (ge & 1023))
    cidx_ref[...] = jnp.where(real, dst_blk * L + pos_blk, 0)


def _post_body(t0_ref, t1_ref, bx_ref, p_ref, pt_ref, out_ref):
    t = t0_ref[0] + t1_ref[0]
    s = bx_ref[...] * t
    num = jnp.dot(s, p_ref[...], preferred_element_type=jnp.float32)
    cnt = t[:, 15:16]
    lik = num / jnp.maximum(cnt, 1.0)
    likb = jnp.dot(lik, pt_ref[...], preferred_element_type=jnp.float32)
    post = s / (likb + 1e-16)
    out_ref[...] = jnp.concatenate(
        [post[:, :CG], jnp.log(lik), jnp.zeros((post.shape[0], 4),
                                               jnp.float32)], axis=1)


def _sc_edge_kernel(nblk0, nblk1, src_hbm, cidx_hbm, wtab_hbm, out_hbm,
                    sa, ca, sb, cb, rows_a, rows_b, zbuf,
                    sem_a, sem_b, sem_z, acc):
    cid = lax.axis_index("c")
    sid = lax.axis_index("s")
    nblk_c = jnp.where(cid == 0, nblk0, nblk1)
    r_base = jnp.where(cid == 0, sid * (nblk0 * RPB),
                       NS * (nblk0 * RPB) + sid * (nblk1 * RPB))

    def fetch(b, sbuf, cbuf):
        r0 = r_base + b * RPB
        pltpu.sync_copy(src_hbm.at[pl.ds(r0, RPB)], sbuf)
        pltpu.sync_copy(cidx_hbm.at[pl.ds(r0, RPB)], cbuf)

    def fire(cbuf, rbuf, sem):
        for j in range(RPB):
            pltpu.async_copy(wtab_hbm.at[cbuf.at[j]],
                             rbuf.at[pl.ds(j * 128, 128)], sem)

    def drain(cbuf, rbuf, sem):
        for j in range(RPB):
            pltpu.make_async_copy(wtab_hbm.at[cbuf.at[j]],
                                  rbuf.at[pl.ds(j * 128, 128)], sem).wait()

    def scat(rbuf, sbuf):
        for j in range(RPB):
            pltpu.sync_copy(rbuf.at[pl.ds(j * 128, 128)],
                            acc.at[sbuf.at[j]], add=True)

    @pl.loop(0, ZROWS)
    def _zb(i):
        zbuf[i, :] = jnp.zeros((LANES,), jnp.float32)
    for t in range(STRIPE // ZROWS):
        pltpu.async_copy(zbuf, acc.at[pl.ds(sid * STRIPE + t * ZROWS, ZROWS)],
                         sem_z)

    fetch(0, sa, ca)
    fire(ca, rows_a, sem_a)
    fetch(1, sb, cb)

    for t in range(STRIPE // ZROWS):
        pltpu.make_async_copy(zbuf,
                              acc.at[pl.ds(sid * STRIPE + t * ZROWS, ZROWS)],
                              sem_z).wait()
    plsc.subcore_barrier()

    @pl.loop(0, nblk_c // 2 - 1)
    def _pipe(h):
        b = 2 * h
        fire(cb, rows_b, sem_b)
        drain(ca, rows_a, sem_a)
        scat(rows_a, sa)
        fetch(b + 2, sa, ca)
        fire(ca, rows_a, sem_a)
        drain(cb, rows_b, sem_b)
        scat(rows_b, sb)
        fetch(b + 3, sb, cb)

    fire(cb, rows_b, sem_b)
    drain(ca, rows_a, sem_a)
    scat(rows_a, sa)
    drain(cb, rows_b, sem_b)
    scat(rows_b, sb)

    plsc.subcore_barrier()
    pltpu.sync_copy(acc.at[pl.ds(sid * STRIPE, STRIPE)],
                    out_hbm.at[cid, pl.ds(sid * STRIPE, STRIPE)])


def kernel(x, prev_h, edge_index, pos, Q_neigh, B):
    Qs = jax.nn.softmax(Q_neigh.astype(jnp.float32), axis=0)
    Bs = jax.nn.softmax(B.astype(jnp.float32), axis=1)
    a4 = jnp.einsum('ijpg,gh->jgpih', Qs, jnp.eye(G, dtype=jnp.float32))
    a = jnp.zeros((LANES, L, LANES), jnp.float32)
    a = a.at[:CG, :, :CG].set(a4.reshape(CG, L, CG)).reshape(LANES, L * LANES)
    bias = jnp.zeros((L, LANES), jnp.float32).at[:, 15].set(1.0)
    bias = bias.reshape(1, L * LANES)
    bst = jnp.zeros((M, LANES), jnp.float32)
    bst = bst.at[:, :CG].set(jnp.transpose(Bs, (1, 0, 2)).reshape(M, CG))
    p_mat = jnp.zeros((LANES, G), jnp.float32)
    p_mat = p_mat.at[jnp.arange(CG), jnp.arange(CG) % G].set(1.0)
    pt_mat = p_mat.T

    a10 = a[:CG]
    ph10 = prev_h.reshape(N, CG).astype(jnp.float32)
    x2 = x.astype(jnp.int32).reshape(N, 1)

    e = edge_index.shape[1]
    assert e % 128 == 0
    blk_total = NW * EDGE_BLK
    nblk = -(-e // blk_total)
    if nblk % 2:
        nblk += 1
    e_pad = nblk * blk_total
    nrows = e_pad // 128
    erows_blk = nrows // NB_TC
    eblk = erows_blk * 128
    ei2 = edge_index.astype(jnp.int32)
    pos1 = pos.astype(jnp.int32)

    wtab, bx, src_p, cidx = pl.pallas_call(
        functools.partial(_pre_body, e, erows_blk),
        grid=(NB_TC,),
        in_specs=[
            pl.BlockSpec((NB_ROWS, CG), lambda i: (i, 0)),
            pl.BlockSpec((CG, L * LANES), lambda i: (0, 0)),
            pl.BlockSpec((1, L * LANES), lambda i: (0, 0)),
            pl.BlockSpec((NB_ROWS, 1), lambda i: (i, 0)),
            pl.BlockSpec((M, LANES), lambda i: (0, 0)),
            pl.BlockSpec((2, eblk), lambda i: (0, i)),
            pl.BlockSpec((eblk,), lambda i: (i,)),
        ],
        out_specs=[
            pl.BlockSpec((NB_ROWS * L, LANES), lambda i: (i, 0)),
            pl.BlockSpec((NB_ROWS, LANES), lambda i: (i, 0)),
            pl.BlockSpec((erows_blk, 128), lambda i: (i, 0)),
            pl.BlockSpec((erows_blk, 128), lambda i: (i, 0)),
        ],
        out_shape=[
            jax.ShapeDtypeStruct((N * L, LANES), jnp.float32),
            jax.ShapeDtypeStruct((N, LANES), jnp.float32),
            jax.ShapeDtypeStruct((nrows, 128), jnp.int32),
            jax.ShapeDtypeStruct((nrows, 128), jnp.int32),
        ],
    )(ph10, a10, bias, x2, bst, ei2, pos1)

    nblk_pair = 2 * nblk
    nblk0 = max(2, int(round(nblk_pair * 0.7 / 2)) * 2)
    nblk1 = nblk_pair - nblk0
    mesh = plsc.VectorSubcoreMesh(core_axis_name="c", subcore_axis_name="s")
    sc_edge = pl.kernel(
        functools.partial(_sc_edge_kernel, nblk0, nblk1),
        out_type=jax.ShapeDtypeStruct((NC, ACC_ROWS, LANES), jnp.float32),
        mesh=mesh,
        scratch_types=[
            pltpu.VMEM((RPB, 128), jnp.int32),
            pltpu.VMEM((RPB, 128), jnp.int32),
            pltpu.VMEM((RPB, 128), jnp.int32),
            pltpu.VMEM((RPB, 128), jnp.int32),
            pltpu.VMEM((EDGE_BLK, LANES), jnp.float32),
            pltpu.VMEM((EDGE_BLK, LANES), jnp.float32),
            pltpu.VMEM((ZROWS, LANES), jnp.float32),
            pltpu.SemaphoreType.DMA,
            pltpu.SemaphoreType.DMA,
            pltpu.SemaphoreType.DMA,
            pltpu.VMEM_SHARED((ACC_ROWS, LANES), jnp.float32),
        ],
        compiler_params=pltpu.CompilerParams(use_tc_tiling_on_sc=False),
    )
    t_part = sc_edge(src_p, cidx, wtab)

    packed = pl.pallas_call(
        _post_body,
        grid=(NB_TC,),
        in_specs=[
            pl.BlockSpec((1, NB_ROWS, LANES), lambda i: (0, i, 0)),
            pl.BlockSpec((1, NB_ROWS, LANES), lambda i: (1, i, 0)),
            pl.BlockSpec((NB_ROWS, LANES), lambda i: (i, 0)),
            pl.BlockSpec((LANES, G), lambda i: (0, 0)),
            pl.BlockSpec((G, LANES), lambda i: (0, 0)),
        ],
        out_specs=pl.BlockSpec((NB_ROWS, LANES), lambda i: (i, 0)),
        out_shape=jax.ShapeDtypeStruct((N, LANES), jnp.float32),
    )(t_part, t_part, bx, p_mat, pt_mat)

    lik_out = packed[:, CG:CG + G]
    return lik_out, packed[:, :CG].reshape(N, C, G)


# speedup vs baseline: 1.1695x; 1.1695x over previous
import functools

import jax
import jax.numpy as jnp
from jax import lax
from jax.experimental import pallas as pl
from jax.experimental.pallas import tpu as pltpu
from jax.experimental.pallas import tpu_sc as plsc

N = 50000
C = 5
G = 2
L = 4
M = 32
CG = C * G
LANES = 16
NC, NS = 2, 16
NW = NC * NS
EDGE_BLK = 1280
RPB = EDGE_BLK // 128
ACC_ROWS = 51200
STRIPE = ACC_ROWS // NS
ZROWS = 640
NB_TC = 10
NB_ROWS = N // NB_TC


def _pre_body(e, erows_blk, ph_ref, a_ref, bias_ref, x_ref, bst_ref,
              ei_ref, pos_ref, wtab_ref, bx_ref,
              srcp_ref, cidx_ref):
    w = jnp.dot(ph_ref[...], a_ref[...], preferred_element_type=jnp.float32)
    wtab_ref[...] = w + bias_ref[...]
    onehot = (x_ref[...] == lax.broadcasted_iota(jnp.int32, (1, M), 1)
              ).astype(jnp.float32)
    bx_ref[...] = jnp.dot(onehot, bst_ref[...],
                          preferred_element_type=jnp.float32)
    i = pl.program_id(0)
    shape = (erows_blk, 128)
    src_blk = ei_ref[0].reshape(shape)
    dst_blk = ei_ref[1].reshape(shape)
    pos_blk = pos_ref[...].reshape(shape)
    ge = ((i * erows_blk + lax.broadcasted_iota(jnp.int32, shape, 0)) * 128
          + lax.broadcasted_iota(jnp.int32, shape, 1))
    real = ge < e
    srcp_ref[...] = jnp.where(real, src_blk, N + (ge & 1023))
    cidx_ref[...] = jnp.where(real, dst_blk * L + pos_blk, 0)


def _post_body(t0_ref, t1_ref, bx_ref, p_ref, pt_ref, out_ref):
    t = t0_ref[0] + t1_ref[0]
    s = bx_ref[...] * t
    num = jnp.dot(s, p_ref[...], preferred_element_type=jnp.float32)
    cnt = t[:, 15:16]
    lik = num / jnp.maximum(cnt, 1.0)
    likb = jnp.dot(lik, pt_ref[...], preferred_element_type=jnp.float32)
    post = s / (likb + 1e-16)
    out_ref[...] = jnp.concatenate(
        [post[:, :CG], jnp.log(lik), jnp.zeros((post.shape[0], 4),
                                               jnp.float32)], axis=1)


def _sc_edge_kernel(nblk0, nblk1, src_hbm, cidx_hbm, wtab_hbm, out_hbm,
                    sa, ca, sb, cb, rows_a, rows_b, zbuf,
                    sem_a, sem_b, sem_z, acc):
    cid = lax.axis_index("c")
    sid = lax.axis_index("s")
    nblk_c = jnp.where(cid == 0, nblk0, nblk1)
    r_base = jnp.where(cid == 0, sid * (nblk0 * RPB),
                       NS * (nblk0 * RPB) + sid * (nblk1 * RPB))

    def fetch(b, sbuf, cbuf):
        r0 = r_base + b * RPB
        pltpu.sync_copy(src_hbm.at[pl.ds(r0, RPB)], sbuf)
        pltpu.sync_copy(cidx_hbm.at[pl.ds(r0, RPB)], cbuf)

    def fire(cbuf, rbuf, sem):
        for j in range(RPB):
            pltpu.async_copy(wtab_hbm.at[cbuf.at[j]],
                             rbuf.at[pl.ds(j * 128, 128)], sem)

    def drain(cbuf, rbuf, sem):
        for j in range(RPB):
            pltpu.make_async_copy(wtab_hbm.at[cbuf.at[j]],
                                  rbuf.at[pl.ds(j * 128, 128)], sem).wait()

    def scat(rbuf, sbuf):
        for j in range(RPB):
            pltpu.sync_copy(rbuf.at[pl.ds(j * 128, 128)],
                            acc.at[sbuf.at[j]], add=True)

    @pl.loop(0, ZROWS)
    def _zb(i):
        zbuf[i, :] = jnp.zeros((LANES,), jnp.float32)
    for t in range(STRIPE // ZROWS):
        pltpu.async_copy(zbuf, acc.at[pl.ds(sid * STRIPE + t * ZROWS, ZROWS)],
                         sem_z)

    fetch(0, sa, ca)
    fire(ca, rows_a, sem_a)
    fetch(1, sb, cb)

    for t in range(STRIPE // ZROWS):
        pltpu.make_async_copy(zbuf,
                              acc.at[pl.ds(sid * STRIPE + t * ZROWS, ZROWS)],
                              sem_z).wait()
    plsc.subcore_barrier()

    @pl.loop(0, nblk_c // 2 - 1)
    def _pipe(h):
        b = 2 * h
        fire(cb, rows_b, sem_b)
        drain(ca, rows_a, sem_a)
        scat(rows_a, sa)
        fetch(b + 2, sa, ca)
        fire(ca, rows_a, sem_a)
        drain(cb, rows_b, sem_b)
        scat(rows_b, sb)
        fetch(b + 3, sb, cb)

    fire(cb, rows_b, sem_b)
    drain(ca, rows_a, sem_a)
    scat(rows_a, sa)
    drain(cb, rows_b, sem_b)
    scat(rows_b, sb)

    plsc.subcore_barrier()
    pltpu.sync_copy(acc.at[pl.ds(sid * STRIPE, STRIPE)],
                    out_hbm.at[cid, pl.ds(sid * STRIPE, STRIPE)])


def kernel(x, prev_h, edge_index, pos, Q_neigh, B):
    Qs = jax.nn.softmax(Q_neigh.astype(jnp.float32), axis=0)
    Bs = jax.nn.softmax(B.astype(jnp.float32), axis=1)
    a4 = jnp.einsum('ijpg,gh->jgpih', Qs, jnp.eye(G, dtype=jnp.float32))
    a = jnp.zeros((LANES, L, LANES), jnp.float32)
    a = a.at[:CG, :, :CG].set(a4.reshape(CG, L, CG)).reshape(LANES, L * LANES)
    bias = jnp.zeros((L, LANES), jnp.float32).at[:, 15].set(1.0)
    bias = bias.reshape(1, L * LANES)
    bst = jnp.zeros((M, LANES), jnp.float32)
    bst = bst.at[:, :CG].set(jnp.transpose(Bs, (1, 0, 2)).reshape(M, CG))
    p_mat = jnp.zeros((LANES, G), jnp.float32)
    p_mat = p_mat.at[jnp.arange(CG), jnp.arange(CG) % G].set(1.0)
    pt_mat = p_mat.T

    a10 = a[:CG]
    ph10 = prev_h.reshape(N, CG).astype(jnp.float32)
    x2 = x.astype(jnp.int32).reshape(N, 1)

    e = edge_index.shape[1]
    assert e % 128 == 0
    blk_total = NW * EDGE_BLK
    nblk = -(-e // blk_total)
    if nblk % 2:
        nblk += 1
    e_pad = nblk * blk_total
    nrows = e_pad // 128
    erows_blk = nrows // NB_TC
    eblk = erows_blk * 128
    ei2 = edge_index.astype(jnp.int32)
    pos1 = pos.astype(jnp.int32)

    wtab, bx, src_p, cidx = pl.pallas_call(
        functools.partial(_pre_body, e, erows_blk),
        grid=(NB_TC,),
        in_specs=[
            pl.BlockSpec((NB_ROWS, CG), lambda i: (i, 0)),
            pl.BlockSpec((CG, L * LANES), lambda i: (0, 0)),
            pl.BlockSpec((1, L * LANES), lambda i: (0, 0)),
            pl.BlockSpec((NB_ROWS, 1), lambda i: (i, 0)),
            pl.BlockSpec((M, LANES), lambda i: (0, 0)),
            pl.BlockSpec((2, eblk), lambda i: (0, i)),
            pl.BlockSpec((eblk,), lambda i: (i,)),
        ],
        out_specs=[
            pl.BlockSpec((NB_ROWS, L * LANES), lambda i: (i, 0)),
            pl.BlockSpec((NB_ROWS, LANES), lambda i: (i, 0)),
            pl.BlockSpec((erows_blk, 128), lambda i: (i, 0)),
            pl.BlockSpec((erows_blk, 128), lambda i: (i, 0)),
        ],
        out_shape=[
            jax.ShapeDtypeStruct((N, L * LANES), jnp.float32),
            jax.ShapeDtypeStruct((N, LANES), jnp.float32),
            jax.ShapeDtypeStruct((nrows, 128), jnp.int32),
            jax.ShapeDtypeStruct((nrows, 128), jnp.int32),
        ],
    )(ph10, a10, bias, x2, bst, ei2, pos1)
    wtab = wtab.reshape(N * L, LANES)

    nblk_pair = 2 * nblk
    nblk0 = max(2, int(round(nblk_pair * 0.7 / 2)) * 2)
    nblk1 = nblk_pair - nblk0
    mesh = plsc.VectorSubcoreMesh(core_axis_name="c", subcore_axis_name="s")
    sc_edge = pl.kernel(
        functools.partial(_sc_edge_kernel, nblk0, nblk1),
        out_type=jax.ShapeDtypeStruct((NC, ACC_ROWS, LANES), jnp.float32),
        mesh=mesh,
        scratch_types=[
            pltpu.VMEM((RPB, 128), jnp.int32),
            pltpu.VMEM((RPB, 128), jnp.int32),
            pltpu.VMEM((RPB, 128), jnp.int32),
            pltpu.VMEM((RPB, 128), jnp.int32),
            pltpu.VMEM((EDGE_BLK, LANES), jnp.float32),
            pltpu.VMEM((EDGE_BLK, LANES), jnp.float32),
            pltpu.VMEM((ZROWS, LANES), jnp.float32),
            pltpu.SemaphoreType.DMA,
            pltpu.SemaphoreType.DMA,
            pltpu.SemaphoreType.DMA,
            pltpu.VMEM_SHARED((ACC_ROWS, LANES), jnp.float32),
        ],
        compiler_params=pltpu.CompilerParams(use_tc_tiling_on_sc=False),
    )
    t_part = sc_edge(src_p, cidx, wtab)

    packed = pl.pallas_call(
        _post_body,
        grid=(NB_TC,),
        in_specs=[
            pl.BlockSpec((1, NB_ROWS, LANES), lambda i: (0, i, 0)),
            pl.BlockSpec((1, NB_ROWS, LANES), lambda i: (1, i, 0)),
            pl.BlockSpec((NB_ROWS, LANES), lambda i: (i, 0)),
            pl.BlockSpec((LANES, G), lambda i: (0, 0)),
            pl.BlockSpec((G, LANES), lambda i: (0, 0)),
        ],
        out_specs=pl.BlockSpec((NB_ROWS, LANES), lambda i: (i, 0)),
        out_shape=jax.ShapeDtypeStruct((N, LANES), jnp.float32),
    )(t_part, t_part, bx, p_mat, pt_mat)

    lik_out = packed[:, CG:CG + G]
    return lik_out, packed[:, :CG].reshape(N, C, G)


# speedup vs baseline: 1.2144x; 1.0384x over previous
import functools

import jax
import jax.numpy as jnp
from jax import lax
from jax.experimental import pallas as pl
from jax.experimental.pallas import tpu as pltpu
from jax.experimental.pallas import tpu_sc as plsc

N = 50000
C = 5
G = 2
L = 4
M = 32
CG = C * G
LANES = 16
NC, NS = 2, 16
NW = NC * NS
EDGE_BLK = 1280
RPB = EDGE_BLK // 128
ACC_ROWS = 51200
STRIPE = ACC_ROWS // NS
ZROWS = 640
NB_TC = 5
NB_ROWS = N // NB_TC


def _pre_body(e, erows_blk, ph_ref, a_ref, bias_ref, x_ref, bst_ref,
              ei_ref, pos_ref, wtab_ref, bx_ref,
              srcp_ref, cidx_ref):
    w = jnp.dot(ph_ref[...], a_ref[...], preferred_element_type=jnp.float32)
    wtab_ref[...] = w + bias_ref[...]
    onehot = (x_ref[...] == lax.broadcasted_iota(jnp.int32, (1, M), 1)
              ).astype(jnp.float32)
    bx_ref[...] = jnp.dot(onehot, bst_ref[...],
                          preferred_element_type=jnp.float32)
    i = pl.program_id(0)
    shape = (erows_blk, 128)
    src_blk = ei_ref[0].reshape(shape)
    dst_blk = ei_ref[1].reshape(shape)
    pos_blk = pos_ref[...].reshape(shape)
    ge = ((i * erows_blk + lax.broadcasted_iota(jnp.int32, shape, 0)) * 128
          + lax.broadcasted_iota(jnp.int32, shape, 1))
    real = ge < e
    srcp_ref[...] = jnp.where(real, src_blk, N + (ge & 1023))
    cidx_ref[...] = jnp.where(real, dst_blk * L + pos_blk, 0)


def _post_body(t0_ref, t1_ref, bx_ref, p_ref, pt_ref, lik_ref, post_ref):
    t = t0_ref[0] + t1_ref[0]
    s = bx_ref[...] * t
    num = jnp.dot(s, p_ref[...], preferred_element_type=jnp.float32)
    cnt = t[:, 15:16]
    lik = num / jnp.maximum(cnt, 1.0)
    lik_ref[...] = jnp.log(lik)
    likb = jnp.dot(lik, pt_ref[...], preferred_element_type=jnp.float32)
    post_ref[...] = (s / (likb + 1e-16))[:, :CG]


def _sc_edge_kernel(nblk0, nblk1, src_hbm, cidx_hbm, wtab_hbm, out_hbm,
                    sa, ca, sb, cb, rows_a, rows_b, zbuf,
                    sem_a, sem_b, sem_z, acc):
    cid = lax.axis_index("c")
    sid = lax.axis_index("s")
    nblk_c = jnp.where(cid == 0, nblk0, nblk1)
    r_base = jnp.where(cid == 0, sid * (nblk0 * RPB),
                       NS * (nblk0 * RPB) + sid * (nblk1 * RPB))

    def fetch(b, sbuf, cbuf):
        r0 = r_base + b * RPB
        pltpu.sync_copy(src_hbm.at[pl.ds(r0, RPB)], sbuf)
        pltpu.sync_copy(cidx_hbm.at[pl.ds(r0, RPB)], cbuf)

    def fire(cbuf, rbuf, sem):
        for j in range(RPB):
            pltpu.async_copy(wtab_hbm.at[cbuf.at[j]],
                             rbuf.at[pl.ds(j * 128, 128)], sem)

    def drain(cbuf, rbuf, sem):
        for j in range(RPB):
            pltpu.make_async_copy(wtab_hbm.at[cbuf.at[j]],
                                  rbuf.at[pl.ds(j * 128, 128)], sem).wait()

    def scat(rbuf, sbuf):
        for j in range(RPB):
            pltpu.sync_copy(rbuf.at[pl.ds(j * 128, 128)],
                            acc.at[sbuf.at[j]], add=True)

    @pl.loop(0, ZROWS)
    def _zb(i):
        zbuf[i, :] = jnp.zeros((LANES,), jnp.float32)
    for t in range(STRIPE // ZROWS):
        pltpu.async_copy(zbuf, acc.at[pl.ds(sid * STRIPE + t * ZROWS, ZROWS)],
                         sem_z)

    fetch(0, sa, ca)
    fire(ca, rows_a, sem_a)
    fetch(1, sb, cb)

    for t in range(STRIPE // ZROWS):
        pltpu.make_async_copy(zbuf,
                              acc.at[pl.ds(sid * STRIPE + t * ZROWS, ZROWS)],
                              sem_z).wait()
    plsc.subcore_barrier()

    @pl.loop(0, nblk_c // 2 - 1)
    def _pipe(h):
        b = 2 * h
        fire(cb, rows_b, sem_b)
        drain(ca, rows_a, sem_a)
        scat(rows_a, sa)
        fetch(b + 2, sa, ca)
        fire(ca, rows_a, sem_a)
        drain(cb, rows_b, sem_b)
        scat(rows_b, sb)
        fetch(b + 3, sb, cb)

    fire(cb, rows_b, sem_b)
    drain(ca, rows_a, sem_a)
    scat(rows_a, sa)
    drain(cb, rows_b, sem_b)
    scat(rows_b, sb)

    plsc.subcore_barrier()
    pltpu.sync_copy(acc.at[pl.ds(sid * STRIPE, STRIPE)],
                    out_hbm.at[cid, pl.ds(sid * STRIPE, STRIPE)])


def kernel(x, prev_h, edge_index, pos, Q_neigh, B):
    Qs = jax.nn.softmax(Q_neigh.astype(jnp.float32), axis=0)
    Bs = jax.nn.softmax(B.astype(jnp.float32), axis=1)
    a4 = jnp.einsum('ijpg,gh->jgpih', Qs, jnp.eye(G, dtype=jnp.float32))
    a = jnp.zeros((LANES, L, LANES), jnp.float32)
    a = a.at[:CG, :, :CG].set(a4.reshape(CG, L, CG)).reshape(LANES, L * LANES)
    bias = jnp.zeros((L, LANES), jnp.float32).at[:, 15].set(1.0)
    bias = bias.reshape(1, L * LANES)
    bst = jnp.zeros((M, LANES), jnp.float32)
    bst = bst.at[:, :CG].set(jnp.transpose(Bs, (1, 0, 2)).reshape(M, CG))
    p_mat = jnp.zeros((LANES, G), jnp.float32)
    p_mat = p_mat.at[jnp.arange(CG), jnp.arange(CG) % G].set(1.0)
    pt_mat = p_mat.T

    a10 = a[:CG]
    ph10 = prev_h.reshape(N, CG).astype(jnp.float32)
    x2 = x.astype(jnp.int32).reshape(N, 1)

    e = edge_index.shape[1]
    assert e % 128 == 0
    blk_total = NW * EDGE_BLK
    nblk = -(-e // blk_total)
    if nblk % 2:
        nblk += 1
    e_pad = nblk * blk_total
    nrows = e_pad // 128
    erows_blk = nrows // NB_TC
    eblk = erows_blk * 128
    ei2 = edge_index.astype(jnp.int32)
    pos1 = pos.astype(jnp.int32)

    wtab, bx, src_p, cidx = pl.pallas_call(
        functools.partial(_pre_body, e, erows_blk),
        grid=(NB_TC,),
        in_specs=[
            pl.BlockSpec((NB_ROWS, CG), lambda i: (i, 0)),
            pl.BlockSpec((CG, L * LANES), lambda i: (0, 0)),
            pl.BlockSpec((1, L * LANES), lambda i: (0, 0)),
            pl.BlockSpec((NB_ROWS, 1), lambda i: (i, 0)),
            pl.BlockSpec((M, LANES), lambda i: (0, 0)),
            pl.BlockSpec((2, eblk), lambda i: (0, i)),
            pl.BlockSpec((eblk,), lambda i: (i,)),
        ],
        out_specs=[
            pl.BlockSpec((NB_ROWS, L * LANES), lambda i: (i, 0)),
            pl.BlockSpec((NB_ROWS, LANES), lambda i: (i, 0)),
            pl.BlockSpec((erows_blk, 128), lambda i: (i, 0)),
            pl.BlockSpec((erows_blk, 128), lambda i: (i, 0)),
        ],
        out_shape=[
            jax.ShapeDtypeStruct((N, L * LANES), jnp.float32),
            jax.ShapeDtypeStruct((N, LANES), jnp.float32),
            jax.ShapeDtypeStruct((nrows, 128), jnp.int32),
            jax.ShapeDtypeStruct((nrows, 128), jnp.int32),
        ],
    )(ph10, a10, bias, x2, bst, ei2, pos1)
    wtab = wtab.reshape(N * L, LANES)

    nblk_pair = 2 * nblk
    nblk0 = max(2, int(round(nblk_pair * 0.7 / 2)) * 2)
    nblk1 = nblk_pair - nblk0
    mesh = plsc.VectorSubcoreMesh(core_axis_name="c", subcore_axis_name="s")
    sc_edge = pl.kernel(
        functools.partial(_sc_edge_kernel, nblk0, nblk1),
        out_type=jax.ShapeDtypeStruct((NC, ACC_ROWS, LANES), jnp.float32),
        mesh=mesh,
        scratch_types=[
            pltpu.VMEM((RPB, 128), jnp.int32),
            pltpu.VMEM((RPB, 128), jnp.int32),
            pltpu.VMEM((RPB, 128), jnp.int32),
            pltpu.VMEM((RPB, 128), jnp.int32),
            pltpu.VMEM((EDGE_BLK, LANES), jnp.float32),
            pltpu.VMEM((EDGE_BLK, LANES), jnp.float32),
            pltpu.VMEM((ZROWS, LANES), jnp.float32),
            pltpu.SemaphoreType.DMA,
            pltpu.SemaphoreType.DMA,
            pltpu.SemaphoreType.DMA,
            pltpu.VMEM_SHARED((ACC_ROWS, LANES), jnp.float32),
        ],
        compiler_params=pltpu.CompilerParams(use_tc_tiling_on_sc=False),
    )
    t_part = sc_edge(src_p, cidx, wtab)

    lik_out, post10 = pl.pallas_call(
        _post_body,
        grid=(NB_TC,),
        in_specs=[
            pl.BlockSpec((1, NB_ROWS, LANES), lambda i: (0, i, 0)),
            pl.BlockSpec((1, NB_ROWS, LANES), lambda i: (1, i, 0)),
            pl.BlockSpec((NB_ROWS, LANES), lambda i: (i, 0)),
            pl.BlockSpec((LANES, G), lambda i: (0, 0)),
            pl.BlockSpec((G, LANES), lambda i: (0, 0)),
        ],
        out_specs=[
            pl.BlockSpec((NB_ROWS, G), lambda i: (i, 0)),
            pl.BlockSpec((NB_ROWS, CG), lambda i: (i, 0)),
        ],
        out_shape=[
            jax.ShapeDtypeStruct((N, G), jnp.float32),
            jax.ShapeDtypeStruct((N, CG), jnp.float32),
        ],
    )(t_part, t_part, bx, p_mat, pt_mat)

    return lik_out, post10.reshape(N, C, G)
